# Initial kernel scaffold; baseline (speedup 1.0000x reference)
import jax
import jax.numpy as jnp
from jax.experimental import pallas as pl


def kernel(x, weight):
    raise NotImplementedError("write your pallas kernel here")


# speedup vs baseline: 1.0949x; 1.0949x over previous
import functools

import jax
import jax.numpy as jnp
from jax import lax
from jax.experimental import pallas as pl
from jax.experimental.pallas import tpu as pltpu
from jax.experimental.pallas import tpu_sc as plsc


def _emb_call(n, d, vocab, chunk):
    nc, ns = 2, 16
    nw = nc * ns
    per_w = n // nw
    n_chunks = per_w // chunk
    mesh = plsc.VectorSubcoreMesh(core_axis_name="c", subcore_axis_name="s")

    @functools.partial(
        pl.kernel,
        out_type=jax.ShapeDtypeStruct((n, d), jnp.float32),
        mesh=mesh,
        scratch_types=[
            pltpu.VMEM((chunk,), jnp.int32),
            pltpu.VMEM((chunk, d), jnp.float32),
            pltpu.SemaphoreType.DMA,
        ],
        compiler_params=pltpu.CompilerParams(use_tc_tiling_on_sc=False),
    )
    def emb(idx_hbm, table_hbm, out_hbm, idx_v, rows_v, sem):
        wid = lax.axis_index("s") * nc + lax.axis_index("c")
        base = wid * per_w

        def body(i, carry):
            off = base + i * chunk
            pltpu.sync_copy(idx_hbm.at[pl.ds(off, chunk)], idx_v)
            pltpu.async_copy(table_hbm.at[idx_v], rows_v, sem).wait()
            pltpu.sync_copy(rows_v, out_hbm.at[pl.ds(off, chunk)])
            return carry

        lax.fori_loop(0, n_chunks, body, 0)

    return emb


def kernel(x, weight):
    b, s = x.shape
    v, d = weight.shape
    n = b * s
    out = _emb_call(n, d, v, chunk=1024)(x.reshape(n), weight)
    return out.reshape(b, s, d)


# speedup vs baseline: 1.1103x; 1.0140x over previous
import functools

import jax
import jax.numpy as jnp
from jax import lax
from jax.experimental import pallas as pl
from jax.experimental.pallas import tpu as pltpu
from jax.experimental.pallas import tpu_sc as plsc


def _emb_call(n, d, chunk):
    nc, ns = 2, 16
    nw = nc * ns
    per_w = n // nw
    n_chunks = per_w // chunk
    assert n_chunks % 2 == 0 and per_w % chunk == 0
    n_groups = n_chunks // 2
    mesh = plsc.VectorSubcoreMesh(core_axis_name="c", subcore_axis_name="s")

    @functools.partial(
        pl.kernel,
        out_type=jax.ShapeDtypeStruct((n, d), jnp.float32),
        mesh=mesh,
        scratch_types=[
            pltpu.VMEM((per_w,), jnp.int32),
            pltpu.VMEM((chunk, d), jnp.float32),
            pltpu.VMEM((chunk, d), jnp.float32),
            pltpu.SemaphoreType.DMA,
            pltpu.SemaphoreType.DMA,
            pltpu.SemaphoreType.DMA,
            pltpu.SemaphoreType.DMA,
        ],
        compiler_params=pltpu.CompilerParams(use_tc_tiling_on_sc=False),
    )
    def emb(idx_hbm, table_hbm, out_hbm, idx_v, rows0, rows1, g0, g1, w0, w1):
        wid = lax.axis_index("s") * nc + lax.axis_index("c")
        base = wid * per_w
        rows = (rows0, rows1)
        gsem = (g0, g1)
        wsem = (w0, w1)

        pltpu.sync_copy(idx_hbm.at[pl.ds(base, per_w)], idx_v)

        def gather(i, b):
            pltpu.async_copy(
                table_hbm.at[idx_v.at[pl.ds(i * chunk, chunk)]], rows[b], gsem[b]
            )

        def put(i, b):
            pltpu.async_copy(rows[b], out_hbm.at[pl.ds(base + i * chunk, chunk)], wsem[b])

        def wait_gather(b):
            pltpu.make_async_copy(table_hbm.at[idx_v.at[pl.ds(0, chunk)]], rows[b], gsem[b]).wait()

        def wait_put(b):
            pltpu.make_async_copy(rows[b], out_hbm.at[pl.ds(0, chunk)], wsem[b]).wait()

        gather(0, 0)

        def group(g, carry):
            i0 = g * 2
            wait_gather(0)

            @pl.when(g > 0)
            def _():
                wait_put(1)

            gather(i0 + 1, 1)
            put(i0, 0)
            wait_gather(1)
            wait_put(0)

            @pl.when(g < n_groups - 1)
            def _():
                gather(i0 + 2, 0)

            put(i0 + 1, 1)
            return carry

        lax.fori_loop(0, n_groups, group, 0)
        wait_put(1)

    return emb


def kernel(x, weight):
    b, s = x.shape
    _, d = weight.shape
    n = b * s
    out = _emb_call(n, d, chunk=1280)(x.reshape(n), weight)
    return out.reshape(b, s, d)


# speedup vs baseline: 1.1107x; 1.0004x over previous
import functools

import jax
import jax.numpy as jnp
from jax import lax
from jax.experimental import pallas as pl
from jax.experimental.pallas import tpu as pltpu
from jax.experimental.pallas import tpu_sc as plsc


def _emb_call(n, d, chunk):
    nc, ns = 2, 16
    nw = nc * ns
    per_w = n // nw
    n_chunks = per_w // chunk
    assert n_chunks % 2 == 0 and per_w % chunk == 0
    n_groups = n_chunks // 2
    mesh = plsc.VectorSubcoreMesh(core_axis_name="c", subcore_axis_name="s")

    @functools.partial(
        pl.kernel,
        out_type=jax.ShapeDtypeStruct((n, d), jnp.float32),
        mesh=mesh,
        scratch_types=[
            pltpu.VMEM((per_w,), jnp.int32),
            pltpu.VMEM((chunk, d), jnp.float32),
            pltpu.VMEM((chunk, d), jnp.float32),
            pltpu.SemaphoreType.DMA,
            pltpu.SemaphoreType.DMA,
            pltpu.SemaphoreType.DMA,
            pltpu.SemaphoreType.DMA,
        ],
        compiler_params=pltpu.CompilerParams(use_tc_tiling_on_sc=False),
    )
    def emb(idx_hbm, table_hbm, out_hbm, idx_v, rows0, rows1, g0, g1, w0, w1):
        wid = lax.axis_index("s") * nc + lax.axis_index("c")
        base = wid * per_w
        rows = (rows0, rows1)
        gsem = (g0, g1)
        wsem = (w0, w1)

        pltpu.sync_copy(idx_hbm.at[pl.ds(base, per_w)], idx_v)

        sub = chunk // 8

        def gather(i, b):
            for j in range(8):
                pltpu.async_copy(
                    table_hbm.at[idx_v.at[pl.ds(i * chunk + j * sub, sub)]],
                    rows[b].at[pl.ds(j * sub, sub)],
                    gsem[b],
                )

        def put(i, b):
            pltpu.async_copy(rows[b], out_hbm.at[pl.ds(base + i * chunk, chunk)], wsem[b])

        def wait_gather(b):
            pltpu.make_async_copy(table_hbm.at[idx_v.at[pl.ds(0, chunk)]], rows[b], gsem[b]).wait()

        def wait_put(b):
            pltpu.make_async_copy(rows[b], out_hbm.at[pl.ds(0, chunk)], wsem[b]).wait()

        gather(0, 0)

        def group(g, carry):
            i0 = g * 2
            wait_gather(0)

            @pl.when(g > 0)
            def _():
                wait_put(1)

            gather(i0 + 1, 1)
            put(i0, 0)
            wait_gather(1)
            wait_put(0)

            @pl.when(g < n_groups - 1)
            def _():
                gather(i0 + 2, 0)

            put(i0 + 1, 1)
            return carry

        lax.fori_loop(0, n_groups, group, 0)
        wait_put(1)

    return emb


def kernel(x, weight):
    b, s = x.shape
    _, d = weight.shape
    n = b * s
    out = _emb_call(n, d, chunk=1280)(x.reshape(n), weight)
    return out.reshape(b, s, d)


# speedup vs baseline: 1.7913x; 1.6128x over previous
import functools

import jax
import jax.numpy as jnp
from jax import lax
from jax.experimental import pallas as pl
from jax.experimental.pallas import tpu as pltpu
from jax.experimental.pallas import tpu_sc as plsc


def _emb_call(bsz, seq, n, d, bpc):
    nc, ns = 2, 16
    nw = nc * ns
    b_per_w = bsz // nw
    chunk = bpc * seq
    per_w = b_per_w * seq
    n_chunks = b_per_w // bpc
    assert n_chunks % 2 == 0 and b_per_w % bpc == 0
    n_groups = n_chunks // 2
    mesh = plsc.VectorSubcoreMesh(core_axis_name="c", subcore_axis_name="s")

    @functools.partial(
        pl.kernel,
        out_type=jax.ShapeDtypeStruct((bsz, seq, d), jnp.float32),
        mesh=mesh,
        scratch_types=[
            pltpu.VMEM((b_per_w, seq), jnp.int32),
            pltpu.VMEM((bpc, seq, d), jnp.float32),
            pltpu.VMEM((bpc, seq, d), jnp.float32),
            pltpu.SemaphoreType.DMA,
            pltpu.SemaphoreType.DMA,
            pltpu.SemaphoreType.DMA,
            pltpu.SemaphoreType.DMA,
        ],
        compiler_params=pltpu.CompilerParams(use_tc_tiling_on_sc=False),
    )
    def emb(x_hbm, table_hbm, out3_hbm, idx_v, rows0, rows1, g0, g1, w0, w1):
        wid = lax.axis_index("s") * nc + lax.axis_index("c")
        base = wid * b_per_w
        rows = (rows0, rows1)
        gsem = (g0, g1)
        wsem = (w0, w1)

        pltpu.sync_copy(x_hbm.at[pl.ds(wid * b_per_w, b_per_w)], idx_v)

        def gather(i, b):
            for j in range(bpc):
                pltpu.async_copy(
                    table_hbm.at[idx_v.at[i * bpc + j]],
                    rows[b].at[j],
                    gsem[b],
                )

        def put(i, b):
            pltpu.async_copy(rows[b], out3_hbm.at[pl.ds(base + i * bpc, bpc)], wsem[b])

        def wait_gather(b):
            for j in range(bpc):
                pltpu.make_async_copy(
                    table_hbm.at[idx_v.at[0]], rows[b].at[j], gsem[b]
                ).wait()

        def wait_put(b):
            pltpu.make_async_copy(rows[b], out3_hbm.at[pl.ds(0, bpc)], wsem[b]).wait()

        gather(0, 0)

        def group(g, carry):
            i0 = g * 2
            wait_gather(0)

            @pl.when(g > 0)
            def _():
                wait_put(1)

            gather(i0 + 1, 1)
            put(i0, 0)
            wait_gather(1)
            wait_put(0)

            @pl.when(g < n_groups - 1)
            def _():
                gather(i0 + 2, 0)

            put(i0 + 1, 1)
            return carry

        lax.fori_loop(0, n_groups, group, 0)
        wait_put(1)

    return emb


def kernel(x, weight):
    b, s = x.shape
    _, d = weight.shape
    return _emb_call(b, s, b * s, d, bpc=16)(x, weight)
